# Initial kernel scaffold; baseline (speedup 1.0000x reference)
import jax
import jax.numpy as jnp
from jax.experimental import pallas as pl


def kernel(x, edge_index, W1, a1s, a1d, b1, W2, a2s, a2d, b2, W3, a3s, a3d, b3, Wlin, blin):
    raise NotImplementedError("write your pallas kernel here")


# speedup vs baseline: 32.1239x; 32.1239x over previous
import dataclasses
import functools

import jax
import jax.numpy as jnp
from jax import lax
from jax.experimental import pallas as pl
from jax.experimental.pallas import tpu as pltpu
from jax.experimental.pallas import tpu_sc as plsc

N = 10000
E = 320000
F_IN = 128
HID = 16

NPAD = 10240
PADN = N
EPAD = 327680
NW = 32
EPW = EPAD // NW
CH = 128
NCHUNK = EPW // CH
ROWS_PT = NPAD // 16

_BLK = 1024


def _attn_cols(h, a_s_ref, a_d_ref):
    a_s = jnp.sum(h * a_s_ref[...], axis=1, keepdims=True)
    a_d = jnp.sum(h * a_d_ref[...], axis=1, keepdims=True)
    return jnp.concatenate([a_s, a_d], axis=1)


def _tc1_body(x_ref, w_ref, a_s_ref, a_d_ref, h_ref, asad_ref):
    h = jnp.dot(x_ref[...], w_ref[...], preferred_element_type=jnp.float32)
    h_ref[...] = h
    asad_ref[...] = _attn_cols(h, a_s_ref, a_d_ref)


def _tc_layer1(x_pad, W1, a1s, a1d):
    grid = (NPAD // _BLK,)
    return pl.pallas_call(
        _tc1_body,
        grid=grid,
        in_specs=[
            pl.BlockSpec((_BLK, F_IN), lambda i: (i, 0)),
            pl.BlockSpec((F_IN, HID), lambda i: (0, 0)),
            pl.BlockSpec((1, HID), lambda i: (0, 0)),
            pl.BlockSpec((1, HID), lambda i: (0, 0)),
        ],
        out_specs=[
            pl.BlockSpec((_BLK, HID), lambda i: (i, 0)),
            pl.BlockSpec((_BLK, 2), lambda i: (i, 0)),
        ],
        out_shape=[
            jax.ShapeDtypeStruct((NPAD, HID), jnp.float32),
            jax.ShapeDtypeStruct((NPAD, 2), jnp.float32),
        ],
    )(x_pad, W1, a1s, a1d)


def _post_aggregate(hpre_ref, asad_ref, acc_ref, b_ref):
    hpre = hpre_ref[...]
    asad = asad_ref[...]
    es = asad[:, 0:1] + asad[:, 1:2]
    ws = jnp.exp(jnp.where(es > 0, es, 0.2 * es))
    acc0 = acc_ref[0]
    acc1 = acc_ref[1]
    msg = acc0[:, 0:HID] + acc1[:, 0:HID] + ws * hpre
    den = acc0[:, HID:HID + 1] + acc1[:, HID:HID + 1] + ws
    g = msg / den + b_ref[...]
    g = jnp.where(g > 0, g, jnp.exp(jnp.minimum(g, 0.0)) - 1.0)
    nrm = jnp.sqrt(jnp.sum(g * g, axis=1, keepdims=True))
    return g / jnp.maximum(nrm, 1e-12)


def _tcmid_body(hpre_ref, asad_ref, acc_ref, b_ref, w_ref, a_s_ref, a_d_ref,
                hout_ref, hnext_ref, asadn_ref):
    hout = _post_aggregate(hpre_ref, asad_ref, acc_ref, b_ref)
    hout_ref[...] = hout
    hn = jnp.dot(hout, w_ref[...], preferred_element_type=jnp.float32)
    hnext_ref[...] = hn
    asadn_ref[...] = _attn_cols(hn, a_s_ref, a_d_ref)


def _tc_mid(hpre, asad, acc, b, Wn, ans, and_):
    grid = (NPAD // _BLK,)
    return pl.pallas_call(
        _tcmid_body,
        grid=grid,
        in_specs=[
            pl.BlockSpec((_BLK, HID), lambda i: (i, 0)),
            pl.BlockSpec((_BLK, 2), lambda i: (i, 0)),
            pl.BlockSpec((2, _BLK, 32), lambda i: (0, i, 0)),
            pl.BlockSpec((1, HID), lambda i: (0, 0)),
            pl.BlockSpec((HID, HID), lambda i: (0, 0)),
            pl.BlockSpec((1, HID), lambda i: (0, 0)),
            pl.BlockSpec((1, HID), lambda i: (0, 0)),
        ],
        out_specs=[
            pl.BlockSpec((_BLK, HID), lambda i: (i, 0)),
            pl.BlockSpec((_BLK, HID), lambda i: (i, 0)),
            pl.BlockSpec((_BLK, 2), lambda i: (i, 0)),
        ],
        out_shape=[
            jax.ShapeDtypeStruct((NPAD, HID), jnp.float32),
            jax.ShapeDtypeStruct((NPAD, HID), jnp.float32),
            jax.ShapeDtypeStruct((NPAD, 2), jnp.float32),
        ],
    )(hpre, asad, acc, b, Wn, ans, and_)


def _tcfin_body(h1_ref, h2_ref, hpre_ref, asad_ref, acc_ref, b_ref,
                wlin_ref, blin_ref, out_ref):
    h3 = _post_aggregate(hpre_ref, asad_ref, acc_ref, b_ref)
    emb = jnp.concatenate([h1_ref[...], h2_ref[...], h3], axis=1)
    wlin = wlin_ref[...]
    cols = [jnp.sum(emb * wlin[c:c + 1, :], axis=1, keepdims=True)
            for c in range(4)]
    logits = jnp.concatenate(cols, axis=1) + blin_ref[...]
    m = jnp.max(logits, axis=1, keepdims=True)
    lse = m + jnp.log(jnp.sum(jnp.exp(logits - m), axis=1, keepdims=True))
    out_ref[...] = logits - lse


def _tc_final(h1, h2, hpre, asad, acc, b, Wlin, blin):
    grid = (NPAD // _BLK,)
    return pl.pallas_call(
        _tcfin_body,
        grid=grid,
        in_specs=[
            pl.BlockSpec((_BLK, HID), lambda i: (i, 0)),
            pl.BlockSpec((_BLK, HID), lambda i: (i, 0)),
            pl.BlockSpec((_BLK, HID), lambda i: (i, 0)),
            pl.BlockSpec((_BLK, 2), lambda i: (i, 0)),
            pl.BlockSpec((2, _BLK, 32), lambda i: (0, i, 0)),
            pl.BlockSpec((1, HID), lambda i: (0, 0)),
            pl.BlockSpec((4, 3 * HID), lambda i: (0, 0)),
            pl.BlockSpec((1, 4), lambda i: (0, 0)),
        ],
        out_specs=pl.BlockSpec((_BLK, 4), lambda i: (i, 0)),
        out_shape=jax.ShapeDtypeStruct((NPAD, 4), jnp.float32),
    )(h1, h2, hpre, asad, acc, b, Wlin, blin)


def _sc_compiler_params():
    cp = pltpu.CompilerParams()
    if "needs_layout_passes" in pltpu.CompilerParams.__dataclass_fields__:
        cp = dataclasses.replace(cp, needs_layout_passes=False)
    if "use_tc_tiling_on_sc" in pltpu.CompilerParams.__dataclass_fields__:
        cp = dataclasses.replace(cp, use_tc_tiling_on_sc=False)
    return cp


def _sc_edges(h, asad, src, dst):
    mesh = plsc.VectorSubcoreMesh(core_axis_name="c", subcore_axis_name="s")

    @functools.partial(
        pl.kernel,
        out_type=jax.ShapeDtypeStruct((2, NPAD, 32), jnp.float32),
        mesh=mesh,
        compiler_params=_sc_compiler_params(),
        scratch_types=[
            pltpu.VMEM((NPAD, 2), jnp.float32),
            pltpu.VMEM((CH,), jnp.int32),
            pltpu.VMEM((CH,), jnp.int32),
            pltpu.VMEM((CH, HID), jnp.float32),
            pltpu.VMEM((CH, 32), jnp.float32),
            pltpu.VMEM_SHARED((NPAD, 32), jnp.float32),
        ],
    )
    def body(h_hbm, asad_hbm, src_hbm, dst_hbm, out_hbm,
             asad_v, src_v, dst_v, rows_v, srows_v, acc_sh):
        cid = lax.axis_index("c")
        sid = lax.axis_index("s")
        wid = cid * 16 + sid

        pltpu.sync_copy(asad_hbm, asad_v)

        z16 = jnp.zeros((16,), jnp.float32)

        @pl.loop(0, CH)
        def _(r):
            srows_v[r, pl.ds(0, 16)] = z16
            srows_v[r, pl.ds(16, 16)] = z16

        @pl.loop(0, ROWS_PT // CH)
        def _(j):
            pltpu.sync_copy(srows_v, acc_sh.at[pl.ds(sid * ROWS_PT + j * CH, CH)])

        plsc.subcore_barrier()

        iota16 = lax.iota(jnp.int32, 16)
        zero_i = jnp.zeros((16,), jnp.int32)
        one_i = jnp.full((16,), 1, jnp.int32)
        wcol_i = jnp.full((16,), 16, jnp.int32)

        @pl.loop(0, NCHUNK)
        def _(ci):
            base = wid * EPW + ci * CH
            pltpu.sync_copy(src_hbm.at[pl.ds(base, CH)], src_v)
            pltpu.sync_copy(dst_hbm.at[pl.ds(base, CH)], dst_v)
            pltpu.sync_copy(h_hbm.at[src_v], rows_v)

            @pl.loop(0, CH // 16)
            def _(g):
                src16 = src_v[pl.ds(g * 16, 16)]
                dst16 = dst_v[pl.ds(g * 16, 16)]
                a_s = plsc.load_gather(asad_v, [src16, zero_i])
                a_d = plsc.load_gather(asad_v, [dst16, one_i])
                e = a_s + a_d
                e = jnp.where(e > 0, e, 0.2 * e)
                w = jnp.exp(e)
                plsc.store_scatter(srows_v, [g * 16 + iota16, wcol_i], w)
                for l in range(16):
                    r = g * 16 + l
                    srows_v[r, pl.ds(0, 16)] = rows_v[r, :] * w[l]

            pltpu.sync_copy(srows_v, acc_sh.at[dst_v], add=True)

        plsc.subcore_barrier()

        pltpu.sync_copy(acc_sh.at[pl.ds(sid * ROWS_PT, ROWS_PT)],
                        out_hbm.at[cid, pl.ds(sid * ROWS_PT, ROWS_PT)])

    return body(h, asad, src, dst)


def kernel(x, edge_index, W1, a1s, a1d, b1, W2, a2s, a2d, b2,
           W3, a3s, a3d, b3, Wlin, blin):
    x_pad = jnp.pad(x, ((0, NPAD - N), (0, 0)))
    ei = edge_index.astype(jnp.int32)
    pad = jnp.full((EPAD - E,), PADN, jnp.int32)
    src = jnp.concatenate([ei[0], pad])
    dst = jnp.concatenate([ei[1], pad])

    b1r = b1.reshape(1, HID)
    b2r = b2.reshape(1, HID)
    b3r = b3.reshape(1, HID)
    blinr = blin.reshape(1, 4)

    h1pre, asad1 = _tc_layer1(x_pad, W1, a1s, a1d)
    acc1 = _sc_edges(h1pre, asad1, src, dst)
    h1, h2pre, asad2 = _tc_mid(h1pre, asad1, acc1, b1r, W2, a2s, a2d)
    acc2 = _sc_edges(h2pre, asad2, src, dst)
    h2, h3pre, asad3 = _tc_mid(h2pre, asad2, acc2, b2r, W3, a3s, a3d)
    acc3 = _sc_edges(h3pre, asad3, src, dst)
    out = _tc_final(h1, h2, h3pre, asad3, acc3, b3r, Wlin, blinr)
    return out[:N]


# speedup vs baseline: 42.6253x; 1.3269x over previous
import dataclasses
import functools

import jax
import jax.numpy as jnp
from jax import lax
from jax.experimental import pallas as pl
from jax.experimental.pallas import tpu as pltpu
from jax.experimental.pallas import tpu_sc as plsc

N = 10000
E = 320000
F_IN = 128
HID = 16

NPAD = 10240
PADN = N
EPAD = 327680
NW = 32
EPW = EPAD // NW
CH = 128
NCHUNK = EPW // CH
ROWS_PT = NPAD // 16

_BLK = 1024


def _attn_cols(h, a_s_ref, a_d_ref):
    a_s = jnp.sum(h * a_s_ref[...], axis=1, keepdims=True)
    a_d = jnp.sum(h * a_d_ref[...], axis=1, keepdims=True)
    return jnp.concatenate([a_s, a_d], axis=1)


def _tc1_body(x_ref, w_ref, a_s_ref, a_d_ref, h_ref, asad_ref):
    h = jnp.dot(x_ref[...], w_ref[...], preferred_element_type=jnp.float32)
    h_ref[...] = h
    asad_ref[...] = _attn_cols(h, a_s_ref, a_d_ref)


def _tc_layer1(x_pad, W1, a1s, a1d):
    grid = (NPAD // _BLK,)
    return pl.pallas_call(
        _tc1_body,
        grid=grid,
        in_specs=[
            pl.BlockSpec((_BLK, F_IN), lambda i: (i, 0)),
            pl.BlockSpec((F_IN, HID), lambda i: (0, 0)),
            pl.BlockSpec((1, HID), lambda i: (0, 0)),
            pl.BlockSpec((1, HID), lambda i: (0, 0)),
        ],
        out_specs=[
            pl.BlockSpec((_BLK, HID), lambda i: (i, 0)),
            pl.BlockSpec((_BLK, 2), lambda i: (i, 0)),
        ],
        out_shape=[
            jax.ShapeDtypeStruct((NPAD, HID), jnp.float32),
            jax.ShapeDtypeStruct((NPAD, 2), jnp.float32),
        ],
    )(x_pad, W1, a1s, a1d)


def _post_aggregate(hpre_ref, asad_ref, acc_ref, b_ref):
    hpre = hpre_ref[...]
    asad = asad_ref[...]
    es = asad[:, 0:1] + asad[:, 1:2]
    ws = jnp.exp(jnp.where(es > 0, es, 0.2 * es))
    acc0 = acc_ref[0]
    acc1 = acc_ref[1]
    msg = acc0[:, 0:HID] + acc1[:, 0:HID] + ws * hpre
    den = acc0[:, HID:HID + 1] + acc1[:, HID:HID + 1] + ws
    g = msg / den + b_ref[...]
    g = jnp.where(g > 0, g, jnp.exp(jnp.minimum(g, 0.0)) - 1.0)
    nrm = jnp.sqrt(jnp.sum(g * g, axis=1, keepdims=True))
    return g / jnp.maximum(nrm, 1e-12)


def _tcmid_body(hpre_ref, asad_ref, acc_ref, b_ref, w_ref, a_s_ref, a_d_ref,
                hout_ref, hnext_ref, asadn_ref):
    hout = _post_aggregate(hpre_ref, asad_ref, acc_ref, b_ref)
    hout_ref[...] = hout
    hn = jnp.dot(hout, w_ref[...], preferred_element_type=jnp.float32)
    hnext_ref[...] = hn
    asadn_ref[...] = _attn_cols(hn, a_s_ref, a_d_ref)


def _tc_mid(hpre, asad, acc, b, Wn, ans, and_):
    grid = (NPAD // _BLK,)
    return pl.pallas_call(
        _tcmid_body,
        grid=grid,
        in_specs=[
            pl.BlockSpec((_BLK, HID), lambda i: (i, 0)),
            pl.BlockSpec((_BLK, 2), lambda i: (i, 0)),
            pl.BlockSpec((2, _BLK, 32), lambda i: (0, i, 0)),
            pl.BlockSpec((1, HID), lambda i: (0, 0)),
            pl.BlockSpec((HID, HID), lambda i: (0, 0)),
            pl.BlockSpec((1, HID), lambda i: (0, 0)),
            pl.BlockSpec((1, HID), lambda i: (0, 0)),
        ],
        out_specs=[
            pl.BlockSpec((_BLK, HID), lambda i: (i, 0)),
            pl.BlockSpec((_BLK, HID), lambda i: (i, 0)),
            pl.BlockSpec((_BLK, 2), lambda i: (i, 0)),
        ],
        out_shape=[
            jax.ShapeDtypeStruct((NPAD, HID), jnp.float32),
            jax.ShapeDtypeStruct((NPAD, HID), jnp.float32),
            jax.ShapeDtypeStruct((NPAD, 2), jnp.float32),
        ],
    )(hpre, asad, acc, b, Wn, ans, and_)


def _tcfin_body(h1_ref, h2_ref, hpre_ref, asad_ref, acc_ref, b_ref,
                wlin_ref, blin_ref, out_ref):
    h3 = _post_aggregate(hpre_ref, asad_ref, acc_ref, b_ref)
    emb = jnp.concatenate([h1_ref[...], h2_ref[...], h3], axis=1)
    wlin = wlin_ref[...]
    cols = [jnp.sum(emb * wlin[c:c + 1, :], axis=1, keepdims=True)
            for c in range(4)]
    logits = jnp.concatenate(cols, axis=1) + blin_ref[...]
    m = jnp.max(logits, axis=1, keepdims=True)
    lse = m + jnp.log(jnp.sum(jnp.exp(logits - m), axis=1, keepdims=True))
    out_ref[...] = logits - lse


def _tc_final(h1, h2, hpre, asad, acc, b, Wlin, blin):
    grid = (NPAD // _BLK,)
    return pl.pallas_call(
        _tcfin_body,
        grid=grid,
        in_specs=[
            pl.BlockSpec((_BLK, HID), lambda i: (i, 0)),
            pl.BlockSpec((_BLK, HID), lambda i: (i, 0)),
            pl.BlockSpec((_BLK, HID), lambda i: (i, 0)),
            pl.BlockSpec((_BLK, 2), lambda i: (i, 0)),
            pl.BlockSpec((2, _BLK, 32), lambda i: (0, i, 0)),
            pl.BlockSpec((1, HID), lambda i: (0, 0)),
            pl.BlockSpec((4, 3 * HID), lambda i: (0, 0)),
            pl.BlockSpec((1, 4), lambda i: (0, 0)),
        ],
        out_specs=pl.BlockSpec((_BLK, 4), lambda i: (i, 0)),
        out_shape=jax.ShapeDtypeStruct((NPAD, 4), jnp.float32),
    )(h1, h2, hpre, asad, acc, b, Wlin, blin)


def _sc_compiler_params():
    cp = pltpu.CompilerParams()
    if "needs_layout_passes" in pltpu.CompilerParams.__dataclass_fields__:
        cp = dataclasses.replace(cp, needs_layout_passes=False)
    if "use_tc_tiling_on_sc" in pltpu.CompilerParams.__dataclass_fields__:
        cp = dataclasses.replace(cp, use_tc_tiling_on_sc=False)
    return cp


GB = 4
NGRP = NCHUNK // GB
GE = GB * CH


def _sc_edges(h, asad, epairs):
    mesh = plsc.VectorSubcoreMesh(core_axis_name="c", subcore_axis_name="s")

    @functools.partial(
        pl.kernel,
        out_type=jax.ShapeDtypeStruct((2, NPAD, 32), jnp.float32),
        mesh=mesh,
        compiler_params=_sc_compiler_params(),
        scratch_types=[
            pltpu.VMEM((NPAD, 2), jnp.float32),
            pltpu.VMEM((GB, 2, CH), jnp.int32),
            pltpu.VMEM((GE, HID), jnp.float32),
            pltpu.VMEM((GE, 32), jnp.float32),
            pltpu.VMEM_SHARED((NPAD, 32), jnp.float32),
            pltpu.SemaphoreType.DMA,
            pltpu.SemaphoreType.DMA,
        ],
    )
    def body(h_hbm, asad_hbm, ep_hbm, out_hbm,
             asad_v, idx_v, rows_v, srows_v, acc_sh, gsem, ssem):
        cid = lax.axis_index("c")
        sid = lax.axis_index("s")
        wid = cid * 16 + sid

        pltpu.sync_copy(asad_hbm, asad_v)

        z16 = jnp.zeros((16,), jnp.float32)

        @pl.loop(0, GE)
        def _(r):
            srows_v[r, pl.ds(0, 16)] = z16
            srows_v[r, pl.ds(16, 16)] = z16

        @pl.loop(0, ROWS_PT // CH)
        def _(j):
            pltpu.sync_copy(srows_v.at[pl.ds(0, CH)],
                            acc_sh.at[pl.ds(sid * ROWS_PT + j * CH, CH)])

        plsc.subcore_barrier()

        iota16 = lax.iota(jnp.int32, 16)
        zero_i = jnp.zeros((16,), jnp.int32)
        one_i = jnp.full((16,), 1, jnp.int32)
        wcol_i = jnp.full((16,), 16, jnp.int32)

        @pl.loop(0, NGRP)
        def _(gi):
            cb = wid * NCHUNK + gi * GB
            pltpu.sync_copy(ep_hbm.at[pl.ds(cb, GB)], idx_v)
            gds = [pltpu.async_copy(h_hbm.at[idx_v.at[j, 0]],
                                    rows_v.at[pl.ds(j * CH, CH)], gsem)
                   for j in range(GB)]
            for d in gds:
                d.wait()

            @pl.loop(0, GE // 16)
            def _(t):
                j = t // (CH // 16)
                gg = t % (CH // 16)
                o = t * 16
                src16 = idx_v[j, 0, pl.ds(gg * 16, 16)]
                dst16 = idx_v[j, 1, pl.ds(gg * 16, 16)]
                a_s = plsc.load_gather(asad_v, [src16, zero_i])
                a_d = plsc.load_gather(asad_v, [dst16, one_i])
                e = a_s + a_d
                e = jnp.where(e > 0, e, 0.2 * e)
                w = jnp.exp(e)
                plsc.store_scatter(srows_v, [o + iota16, wcol_i], w)
                for l in range(16):
                    srows_v[o + l, pl.ds(0, 16)] = rows_v[o + l, :] * w[l]

            sds = [pltpu.async_copy(srows_v.at[pl.ds(j * CH, CH)],
                                    acc_sh.at[idx_v.at[j, 1]], ssem, add=True)
                   for j in range(GB)]
            for d in sds:
                d.wait()

        plsc.subcore_barrier()

        pltpu.sync_copy(acc_sh.at[pl.ds(sid * ROWS_PT, ROWS_PT)],
                        out_hbm.at[cid, pl.ds(sid * ROWS_PT, ROWS_PT)])

    return body(h, asad, epairs)


def kernel(x, edge_index, W1, a1s, a1d, b1, W2, a2s, a2d, b2,
           W3, a3s, a3d, b3, Wlin, blin):
    x_pad = jnp.pad(x, ((0, NPAD - N), (0, 0)))
    ei = edge_index.astype(jnp.int32)
    pad = jnp.full((EPAD - E,), PADN, jnp.int32)
    src = jnp.concatenate([ei[0], pad])
    dst = jnp.concatenate([ei[1], pad])
    epairs = jnp.stack([src.reshape(-1, CH), dst.reshape(-1, CH)], axis=1)

    b1r = b1.reshape(1, HID)
    b2r = b2.reshape(1, HID)
    b3r = b3.reshape(1, HID)
    blinr = blin.reshape(1, 4)

    h1pre, asad1 = _tc_layer1(x_pad, W1, a1s, a1d)
    acc1 = _sc_edges(h1pre, asad1, epairs)
    h1, h2pre, asad2 = _tc_mid(h1pre, asad1, acc1, b1r, W2, a2s, a2d)
    acc2 = _sc_edges(h2pre, asad2, epairs)
    h2, h3pre, asad3 = _tc_mid(h2pre, asad2, acc2, b2r, W3, a3s, a3d)
    acc3 = _sc_edges(h3pre, asad3, epairs)
    out = _tc_final(h1, h2, h3pre, asad3, acc3, b3r, Wlin, blinr)
    return out[:N]


# speedup vs baseline: 47.3937x; 1.1119x over previous
import dataclasses
import functools

import jax
import jax.numpy as jnp
from jax import lax
from jax.experimental import pallas as pl
from jax.experimental.pallas import tpu as pltpu
from jax.experimental.pallas import tpu_sc as plsc

N = 10000
E = 320000
F_IN = 128
HID = 16

NPAD = 10240
PADN = N
EPAD = 327680
NW = 32
EPW = EPAD // NW
CH = 128
NCHUNK = EPW // CH
ROWS_PT = NPAD // 16
ACCW = 32

_BLK = 1024


def _attn_cols(h, a_s_ref, a_d_ref):
    a_s = jnp.sum(h * a_s_ref[...], axis=1, keepdims=True)
    a_d = jnp.sum(h * a_d_ref[...], axis=1, keepdims=True)
    return jnp.concatenate([a_s, a_d], axis=1)


def _tc1_body(x_ref, w_ref, a_s_ref, a_d_ref, h_ref, asad_ref):
    h = jnp.dot(x_ref[...], w_ref[...], preferred_element_type=jnp.float32)
    h_ref[...] = h
    asad_ref[...] = _attn_cols(h, a_s_ref, a_d_ref)


def _tc_layer1(x_pad, W1, a1s, a1d):
    grid = (NPAD // _BLK,)
    return pl.pallas_call(
        _tc1_body,
        grid=grid,
        in_specs=[
            pl.BlockSpec((_BLK, F_IN), lambda i: (i, 0)),
            pl.BlockSpec((F_IN, HID), lambda i: (0, 0)),
            pl.BlockSpec((1, HID), lambda i: (0, 0)),
            pl.BlockSpec((1, HID), lambda i: (0, 0)),
        ],
        out_specs=[
            pl.BlockSpec((_BLK, HID), lambda i: (i, 0)),
            pl.BlockSpec((_BLK, 2), lambda i: (i, 0)),
        ],
        out_shape=[
            jax.ShapeDtypeStruct((NPAD, HID), jnp.float32),
            jax.ShapeDtypeStruct((NPAD, 2), jnp.float32),
        ],
    )(x_pad, W1, a1s, a1d)


def _post_aggregate(hpre_ref, asad_ref, acc_ref, b_ref):
    hpre = hpre_ref[...]
    asad = asad_ref[...]
    es = asad[:, 0:1] + asad[:, 1:2]
    ws = jnp.exp(jnp.where(es > 0, es, 0.2 * es))
    acc0 = acc_ref[0]
    acc1 = acc_ref[1]
    msg = acc0[:, 0:HID] + acc1[:, 0:HID] + ws * hpre
    den = acc0[:, HID:HID + 1] + acc1[:, HID:HID + 1] + ws
    g = msg / den + b_ref[...]
    g = jnp.where(g > 0, g, jnp.exp(jnp.minimum(g, 0.0)) - 1.0)
    nrm = jnp.sqrt(jnp.sum(g * g, axis=1, keepdims=True))
    return g / jnp.maximum(nrm, 1e-12)


def _tcmid_body(hpre_ref, asad_ref, acc_ref, b_ref, w_ref, a_s_ref, a_d_ref,
                hout_ref, hnext_ref, asadn_ref):
    hout = _post_aggregate(hpre_ref, asad_ref, acc_ref, b_ref)
    hout_ref[...] = hout
    hn = jnp.dot(hout, w_ref[...], preferred_element_type=jnp.float32)
    hnext_ref[...] = hn
    asadn_ref[...] = _attn_cols(hn, a_s_ref, a_d_ref)


def _tc_mid(hpre, asad, acc, b, Wn, ans, and_):
    grid = (NPAD // _BLK,)
    return pl.pallas_call(
        _tcmid_body,
        grid=grid,
        in_specs=[
            pl.BlockSpec((_BLK, HID), lambda i: (i, 0)),
            pl.BlockSpec((_BLK, 2), lambda i: (i, 0)),
            pl.BlockSpec((2, _BLK, ACCW), lambda i: (0, i, 0)),
            pl.BlockSpec((1, HID), lambda i: (0, 0)),
            pl.BlockSpec((HID, HID), lambda i: (0, 0)),
            pl.BlockSpec((1, HID), lambda i: (0, 0)),
            pl.BlockSpec((1, HID), lambda i: (0, 0)),
        ],
        out_specs=[
            pl.BlockSpec((_BLK, HID), lambda i: (i, 0)),
            pl.BlockSpec((_BLK, HID), lambda i: (i, 0)),
            pl.BlockSpec((_BLK, 2), lambda i: (i, 0)),
        ],
        out_shape=[
            jax.ShapeDtypeStruct((NPAD, HID), jnp.float32),
            jax.ShapeDtypeStruct((NPAD, HID), jnp.float32),
            jax.ShapeDtypeStruct((NPAD, 2), jnp.float32),
        ],
    )(hpre, asad, acc, b, Wn, ans, and_)


def _tcfin_body(h1_ref, h2_ref, hpre_ref, asad_ref, acc_ref, b_ref,
                wlin_ref, blin_ref, out_ref):
    h3 = _post_aggregate(hpre_ref, asad_ref, acc_ref, b_ref)
    emb = jnp.concatenate([h1_ref[...], h2_ref[...], h3], axis=1)
    wlin = wlin_ref[...]
    cols = [jnp.sum(emb * wlin[c:c + 1, :], axis=1, keepdims=True)
            for c in range(4)]
    logits = jnp.concatenate(cols, axis=1) + blin_ref[...]
    m = jnp.max(logits, axis=1, keepdims=True)
    lse = m + jnp.log(jnp.sum(jnp.exp(logits - m), axis=1, keepdims=True))
    out_ref[...] = logits - lse


def _tc_final(h1, h2, hpre, asad, acc, b, Wlin, blin):
    grid = (NPAD // _BLK,)
    return pl.pallas_call(
        _tcfin_body,
        grid=grid,
        in_specs=[
            pl.BlockSpec((_BLK, HID), lambda i: (i, 0)),
            pl.BlockSpec((_BLK, HID), lambda i: (i, 0)),
            pl.BlockSpec((_BLK, HID), lambda i: (i, 0)),
            pl.BlockSpec((_BLK, 2), lambda i: (i, 0)),
            pl.BlockSpec((2, _BLK, ACCW), lambda i: (0, i, 0)),
            pl.BlockSpec((1, HID), lambda i: (0, 0)),
            pl.BlockSpec((4, 3 * HID), lambda i: (0, 0)),
            pl.BlockSpec((1, 4), lambda i: (0, 0)),
        ],
        out_specs=pl.BlockSpec((_BLK, 4), lambda i: (i, 0)),
        out_shape=jax.ShapeDtypeStruct((NPAD, 4), jnp.float32),
    )(h1, h2, hpre, asad, acc, b, Wlin, blin)


def _sc_compiler_params():
    cp = pltpu.CompilerParams()
    if "needs_layout_passes" in pltpu.CompilerParams.__dataclass_fields__:
        cp = dataclasses.replace(cp, needs_layout_passes=False)
    if "use_tc_tiling_on_sc" in pltpu.CompilerParams.__dataclass_fields__:
        cp = dataclasses.replace(cp, use_tc_tiling_on_sc=False)
    return cp


GB = 2
NGRP = NCHUNK // GB
GE = GB * CH


def _sc_edges(h, asad, epairs):
    mesh = plsc.VectorSubcoreMesh(core_axis_name="c", subcore_axis_name="s")

    @functools.partial(
        pl.kernel,
        out_type=jax.ShapeDtypeStruct((2, NPAD, ACCW), jnp.float32),
        mesh=mesh,
        compiler_params=_sc_compiler_params(),
        scratch_types=[
            pltpu.VMEM((NPAD, 2), jnp.float32),
            [pltpu.VMEM((GB, 2, CH), jnp.int32) for _ in range(4)],
            pltpu.VMEM((GE, HID), jnp.float32),
            [pltpu.VMEM((GE, ACCW), jnp.float32) for _ in range(2)],
            pltpu.VMEM((GE,), jnp.float32),
            pltpu.VMEM_SHARED((NPAD, ACCW), jnp.float32),
            pltpu.SemaphoreType.DMA,
            pltpu.SemaphoreType.DMA,
            [pltpu.SemaphoreType.DMA for _ in range(2)],
        ],
    )
    def body(h_hbm, asad_hbm, ep_hbm, out_hbm,
             asad_v, idx_v, rows_v, srows_v, wbuf_v, acc_sh,
             isem, gsem, ssem):
        cid = lax.axis_index("c")
        sid = lax.axis_index("s")
        wid = cid * 16 + sid

        pltpu.sync_copy(asad_hbm, asad_v)

        z16 = jnp.zeros((16,), jnp.float32)

        for p in range(2):
            @pl.loop(0, GE)
            def _(r, p=p):
                srows_v[p][r, pl.ds(0, 16)] = z16
                srows_v[p][r, pl.ds(ACCW - 16, 16)] = z16

        @pl.loop(0, ROWS_PT // CH)
        def _(j):
            pltpu.sync_copy(srows_v[0].at[pl.ds(0, CH)],
                            acc_sh.at[pl.ds(sid * ROWS_PT + j * CH, CH)])

        plsc.subcore_barrier()

        iota16 = lax.iota(jnp.int32, 16)
        zero_i = jnp.zeros((16,), jnp.int32)
        one_i = jnp.full((16,), 1, jnp.int32)
        wcol_i = jnp.full((16,), 16, jnp.int32)
        cbase = wid * NCHUNK

        def idx_copy(gi, b):
            return pltpu.make_async_copy(
                ep_hbm.at[pl.ds(cbase + gi * GB, GB)], idx_v[b], isem)

        def gather_copy(b, p, j):
            return pltpu.make_async_copy(
                h_hbm.at[idx_v[b].at[j, 0]],
                rows_v.at[pl.ds(j * CH, CH)], gsem)

        def scatter_copy(b, p, j):
            return pltpu.async_copy(
                srows_v[p].at[pl.ds(j * CH, CH)],
                acc_sh.at[idx_v[b].at[j, 1]], ssem[p], add=True)

        def scatter_wait(b, p, j):
            pltpu.make_async_copy(
                srows_v[p].at[pl.ds(j * CH, CH)],
                acc_sh.at[idx_v[b].at[j, 1]], ssem[p]).wait()

        idx_copy(0, 0).start()

        @pl.loop(0, NGRP // 4)
        def _(s):
            for k in range(4):
                g = s * 4 + k
                b = k
                p = k % 2

                idx_copy(g, b).wait()
                for j in range(GB):
                    gather_copy(b, p, j).start()

                @pl.loop(0, GE // 16)
                def _(t, b=b, p=p):
                    j = t // (CH // 16)
                    gg = t % (CH // 16)
                    src16 = idx_v[b][j, 0, pl.ds(gg * 16, 16)]
                    dst16 = idx_v[b][j, 1, pl.ds(gg * 16, 16)]
                    a_s = plsc.load_gather(asad_v, [src16, zero_i])
                    a_d = plsc.load_gather(asad_v, [dst16, one_i])
                    e = a_s + a_d
                    e = jnp.where(e > 0, e, 0.2 * e)
                    wbuf_v[pl.ds(t * 16, 16)] = jnp.exp(e)

                if k < 2:
                    @pl.when(s > 0)
                    def _(b=b, p=p):
                        for j in range(GB):
                            scatter_wait((b + 2) % 4, p, j)
                else:
                    for j in range(GB):
                        scatter_wait((b + 2) % 4, p, j)

                if k < 3:
                    idx_copy(g + 1, (b + 1) % 4).start()
                else:
                    @pl.when(s < NGRP // 4 - 1)
                    def _(g=g, b=b):
                        idx_copy(g + 1, (b + 1) % 4).start()

                for j in range(GB):
                    gather_copy(b, p, j).wait()

                @pl.loop(0, GE // 16)
                def _(t, b=b, p=p):
                    o = t * 16
                    w = wbuf_v[pl.ds(o, 16)]
                    plsc.store_scatter(srows_v[p], [o + iota16, wcol_i], w)
                    for l in range(16):
                        srows_v[p][o + l, pl.ds(0, 16)] = \
                            rows_v[o + l, :] * w[l]

                for j in range(GB):
                    scatter_copy(b, p, j)

        for j in range(GB):
            scatter_wait(2, 0, j)
        for j in range(GB):
            scatter_wait(3, 1, j)

        plsc.subcore_barrier()

        pltpu.sync_copy(acc_sh.at[pl.ds(sid * ROWS_PT, ROWS_PT)],
                        out_hbm.at[cid, pl.ds(sid * ROWS_PT, ROWS_PT)])

    return body(h, asad, epairs)


def kernel(x, edge_index, W1, a1s, a1d, b1, W2, a2s, a2d, b2,
           W3, a3s, a3d, b3, Wlin, blin):
    x_pad = jnp.pad(x, ((0, NPAD - N), (0, 0)))
    ei = edge_index.astype(jnp.int32)
    pad = jnp.full((EPAD - E,), PADN, jnp.int32)
    src = jnp.concatenate([ei[0], pad])
    dst = jnp.concatenate([ei[1], pad])
    epairs = jnp.stack([src.reshape(-1, CH), dst.reshape(-1, CH)], axis=1)

    b1r = b1.reshape(1, HID)
    b2r = b2.reshape(1, HID)
    b3r = b3.reshape(1, HID)
    blinr = blin.reshape(1, 4)

    h1pre, asad1 = _tc_layer1(x_pad, W1, a1s, a1d)
    acc1 = _sc_edges(h1pre, asad1, epairs)
    h1, h2pre, asad2 = _tc_mid(h1pre, asad1, acc1, b1r, W2, a2s, a2d)
    acc2 = _sc_edges(h2pre, asad2, epairs)
    h2, h3pre, asad3 = _tc_mid(h2pre, asad2, acc2, b2r, W3, a3s, a3d)
    acc3 = _sc_edges(h3pre, asad3, epairs)
    out = _tc_final(h1, h2, h3pre, asad3, acc3, b3r, Wlin, blinr)
    return out[:N]


# speedup vs baseline: 56.9633x; 1.2019x over previous
import dataclasses
import functools

import jax
import jax.numpy as jnp
from jax import lax
from jax.experimental import pallas as pl
from jax.experimental.pallas import tpu as pltpu
from jax.experimental.pallas import tpu_sc as plsc

N = 10000
E = 320000
F_IN = 128
HID = 16

NPAD = 10240
PADN = N
EPAD = 327680
NW = 32
EPW = EPAD // NW
CH = 128
NCHUNK = EPW // CH
ROWS_PT = NPAD // 16
ACCW = 32

_BLK = 1024


def _attn_cols(h, a_s_ref, a_d_ref):
    a_s = jnp.sum(h * a_s_ref[...], axis=1, keepdims=True)
    a_d = jnp.sum(h * a_d_ref[...], axis=1, keepdims=True)
    return jnp.concatenate([a_s, a_d], axis=1)


def _tc1_body(x_ref, w_ref, a_s_ref, a_d_ref, h_ref, asad_ref):
    h = jnp.dot(x_ref[...], w_ref[...], preferred_element_type=jnp.float32)
    h_ref[...] = h
    asad_ref[...] = _attn_cols(h, a_s_ref, a_d_ref)


def _tc_layer1(x_pad, W1, a1s, a1d):
    grid = (NPAD // _BLK,)
    return pl.pallas_call(
        _tc1_body,
        grid=grid,
        in_specs=[
            pl.BlockSpec((_BLK, F_IN), lambda i: (i, 0)),
            pl.BlockSpec((F_IN, HID), lambda i: (0, 0)),
            pl.BlockSpec((1, HID), lambda i: (0, 0)),
            pl.BlockSpec((1, HID), lambda i: (0, 0)),
        ],
        out_specs=[
            pl.BlockSpec((_BLK, HID), lambda i: (i, 0)),
            pl.BlockSpec((_BLK, 2), lambda i: (i, 0)),
        ],
        out_shape=[
            jax.ShapeDtypeStruct((NPAD, HID), jnp.float32),
            jax.ShapeDtypeStruct((NPAD, 2), jnp.float32),
        ],
    )(x_pad, W1, a1s, a1d)


def _post_aggregate(hpre_ref, asad_ref, acc_ref, b_ref):
    hpre = hpre_ref[...]
    asad = asad_ref[...]
    es = asad[:, 0:1] + asad[:, 1:2]
    ws = jnp.exp(jnp.where(es > 0, es, 0.2 * es))
    acc0 = acc_ref[0]
    acc1 = acc_ref[1]
    msg = acc0[:, 0:HID] + acc1[:, 0:HID] + ws * hpre
    den = acc0[:, HID:HID + 1] + acc1[:, HID:HID + 1] + ws
    g = msg / den + b_ref[...]
    g = jnp.where(g > 0, g, jnp.exp(jnp.minimum(g, 0.0)) - 1.0)
    nrm = jnp.sqrt(jnp.sum(g * g, axis=1, keepdims=True))
    return g / jnp.maximum(nrm, 1e-12)


def _tcmid_body(hpre_ref, asad_ref, acc_ref, b_ref, w_ref, a_s_ref, a_d_ref,
                hout_ref, hnext_ref, asadn_ref):
    hout = _post_aggregate(hpre_ref, asad_ref, acc_ref, b_ref)
    hout_ref[...] = hout
    hn = jnp.dot(hout, w_ref[...], preferred_element_type=jnp.float32)
    hnext_ref[...] = hn
    asadn_ref[...] = _attn_cols(hn, a_s_ref, a_d_ref)


def _tc_mid(hpre, asad, acc, b, Wn, ans, and_):
    grid = (NPAD // _BLK,)
    return pl.pallas_call(
        _tcmid_body,
        grid=grid,
        in_specs=[
            pl.BlockSpec((_BLK, HID), lambda i: (i, 0)),
            pl.BlockSpec((_BLK, 2), lambda i: (i, 0)),
            pl.BlockSpec((2, _BLK, ACCW), lambda i: (0, i, 0)),
            pl.BlockSpec((1, HID), lambda i: (0, 0)),
            pl.BlockSpec((HID, HID), lambda i: (0, 0)),
            pl.BlockSpec((1, HID), lambda i: (0, 0)),
            pl.BlockSpec((1, HID), lambda i: (0, 0)),
        ],
        out_specs=[
            pl.BlockSpec((_BLK, HID), lambda i: (i, 0)),
            pl.BlockSpec((_BLK, HID), lambda i: (i, 0)),
            pl.BlockSpec((_BLK, 2), lambda i: (i, 0)),
        ],
        out_shape=[
            jax.ShapeDtypeStruct((NPAD, HID), jnp.float32),
            jax.ShapeDtypeStruct((NPAD, HID), jnp.float32),
            jax.ShapeDtypeStruct((NPAD, 2), jnp.float32),
        ],
    )(hpre, asad, acc, b, Wn, ans, and_)


def _tcfin_body(h1_ref, h2_ref, hpre_ref, asad_ref, acc_ref, b_ref,
                wlin_ref, blin_ref, out_ref):
    h3 = _post_aggregate(hpre_ref, asad_ref, acc_ref, b_ref)
    emb = jnp.concatenate([h1_ref[...], h2_ref[...], h3], axis=1)
    wlin = wlin_ref[...]
    cols = [jnp.sum(emb * wlin[c:c + 1, :], axis=1, keepdims=True)
            for c in range(4)]
    logits = jnp.concatenate(cols, axis=1) + blin_ref[...]
    m = jnp.max(logits, axis=1, keepdims=True)
    lse = m + jnp.log(jnp.sum(jnp.exp(logits - m), axis=1, keepdims=True))
    out_ref[...] = logits - lse


def _tc_final(h1, h2, hpre, asad, acc, b, Wlin, blin):
    grid = (NPAD // _BLK,)
    return pl.pallas_call(
        _tcfin_body,
        grid=grid,
        in_specs=[
            pl.BlockSpec((_BLK, HID), lambda i: (i, 0)),
            pl.BlockSpec((_BLK, HID), lambda i: (i, 0)),
            pl.BlockSpec((_BLK, HID), lambda i: (i, 0)),
            pl.BlockSpec((_BLK, 2), lambda i: (i, 0)),
            pl.BlockSpec((2, _BLK, ACCW), lambda i: (0, i, 0)),
            pl.BlockSpec((1, HID), lambda i: (0, 0)),
            pl.BlockSpec((4, 3 * HID), lambda i: (0, 0)),
            pl.BlockSpec((1, 4), lambda i: (0, 0)),
        ],
        out_specs=pl.BlockSpec((_BLK, 4), lambda i: (i, 0)),
        out_shape=jax.ShapeDtypeStruct((NPAD, 4), jnp.float32),
    )(h1, h2, hpre, asad, acc, b, Wlin, blin)


def _sc_compiler_params():
    cp = pltpu.CompilerParams()
    if "needs_layout_passes" in pltpu.CompilerParams.__dataclass_fields__:
        cp = dataclasses.replace(cp, needs_layout_passes=False)
    if "use_tc_tiling_on_sc" in pltpu.CompilerParams.__dataclass_fields__:
        cp = dataclasses.replace(cp, use_tc_tiling_on_sc=False)
    return cp


GB = 2
NGRP = NCHUNK // GB
GE = GB * CH


def _sc_edges(h, asad, epairs):
    mesh = plsc.VectorSubcoreMesh(core_axis_name="c", subcore_axis_name="s")

    @functools.partial(
        pl.kernel,
        out_type=jax.ShapeDtypeStruct((2, NPAD, ACCW), jnp.float32),
        mesh=mesh,
        compiler_params=_sc_compiler_params(),
        scratch_types=[
            pltpu.VMEM((NPAD, 2), jnp.float32),
            [pltpu.VMEM((GB, 2, CH), jnp.int32) for _ in range(4)],
            pltpu.VMEM((GE, HID), jnp.float32),
            [pltpu.VMEM((GE, ACCW), jnp.float32) for _ in range(2)],
            pltpu.VMEM((GE,), jnp.float32),
            pltpu.VMEM_SHARED((NPAD, ACCW), jnp.float32),
            pltpu.SemaphoreType.DMA,
            pltpu.SemaphoreType.DMA,
            [pltpu.SemaphoreType.DMA for _ in range(2)],
        ],
    )
    def body(h_hbm, asad_hbm, ep_hbm, out_hbm,
             asad_v, idx_v, rows_v, srows_v, wbuf_v, acc_sh,
             isem, gsem, ssem):
        cid = lax.axis_index("c")
        sid = lax.axis_index("s")
        wid = cid * 16 + sid

        pltpu.sync_copy(asad_hbm, asad_v)

        z16 = jnp.zeros((16,), jnp.float32)

        for p in range(2):
            @pl.loop(0, GE)
            def _(r, p=p):
                srows_v[p][r, pl.ds(0, 16)] = z16
                srows_v[p][r, pl.ds(ACCW - 16, 16)] = z16

        @pl.loop(0, ROWS_PT // CH)
        def _(j):
            pltpu.sync_copy(srows_v[0].at[pl.ds(0, CH)],
                            acc_sh.at[pl.ds(sid * ROWS_PT + j * CH, CH)])

        plsc.subcore_barrier()

        iota16 = lax.iota(jnp.int32, 16)
        zero_i = jnp.zeros((16,), jnp.int32)
        one_i = jnp.full((16,), 1, jnp.int32)
        wcol_i = jnp.full((16,), 16, jnp.int32)
        cbase = wid * NCHUNK

        def idx_copy(gi, b):
            return pltpu.make_async_copy(
                ep_hbm.at[pl.ds(cbase + gi * GB, GB)], idx_v[b], isem)

        def gather_copy(b, p, j):
            return pltpu.make_async_copy(
                h_hbm.at[idx_v[b].at[j, 0]],
                rows_v.at[pl.ds(j * CH, CH)], gsem)

        def scatter_copy(b, p, j):
            return pltpu.async_copy(
                srows_v[p].at[pl.ds(j * CH, CH)],
                acc_sh.at[idx_v[b].at[j, 1]], ssem[p], add=True)

        def scatter_wait(b, p, j):
            pltpu.make_async_copy(
                srows_v[p].at[pl.ds(j * CH, CH)],
                acc_sh.at[idx_v[b].at[j, 1]], ssem[p]).wait()

        idx_copy(0, 0).start()

        @pl.loop(0, NGRP // 4)
        def _(s):
            for k in range(4):
                g = s * 4 + k
                b = k
                p = k % 2

                idx_copy(g, b).wait()
                for j in range(GB):
                    gather_copy(b, p, j).start()

                @plsc.parallel_loop(0, GE // 16, unroll=2)
                def _(t, b=b, p=p):
                    j = t // (CH // 16)
                    gg = t % (CH // 16)
                    src16 = idx_v[b][j, 0, pl.ds(gg * 16, 16)]
                    dst16 = idx_v[b][j, 1, pl.ds(gg * 16, 16)]
                    a_s = plsc.load_gather(asad_v, [src16, zero_i])
                    a_d = plsc.load_gather(asad_v, [dst16, one_i])
                    e = a_s + a_d
                    e = jnp.where(e > 0, e, 0.2 * e)
                    wbuf_v[pl.ds(t * 16, 16)] = jnp.exp(e)

                if k < 2:
                    @pl.when(s > 0)
                    def _(b=b, p=p):
                        for j in range(GB):
                            scatter_wait((b + 2) % 4, p, j)
                else:
                    for j in range(GB):
                        scatter_wait((b + 2) % 4, p, j)

                if k < 3:
                    idx_copy(g + 1, (b + 1) % 4).start()
                else:
                    @pl.when(s < NGRP // 4 - 1)
                    def _(g=g, b=b):
                        idx_copy(g + 1, (b + 1) % 4).start()

                for j in range(GB):
                    gather_copy(b, p, j).wait()

                @plsc.parallel_loop(0, GE // 16, unroll=2)
                def _(t, b=b, p=p):
                    o = t * 16
                    w = wbuf_v[pl.ds(o, 16)]
                    plsc.store_scatter(srows_v[p], [o + iota16, wcol_i], w)
                    for l in range(16):
                        wl = lax.gather(
                            w, jnp.full((16, 1), l, jnp.int32),
                            lax.GatherDimensionNumbers(
                                offset_dims=(), collapsed_slice_dims=(0,),
                                start_index_map=(0,)),
                            (1,),
                            mode=lax.GatherScatterMode.PROMISE_IN_BOUNDS)
                        srows_v[p][o + l, pl.ds(0, 16)] = \
                            rows_v[o + l, :] * wl

                for j in range(GB):
                    scatter_copy(b, p, j)

        for j in range(GB):
            scatter_wait(2, 0, j)
        for j in range(GB):
            scatter_wait(3, 1, j)

        plsc.subcore_barrier()

        pltpu.sync_copy(acc_sh.at[pl.ds(sid * ROWS_PT, ROWS_PT)],
                        out_hbm.at[cid, pl.ds(sid * ROWS_PT, ROWS_PT)])

    return body(h, asad, epairs)


def kernel(x, edge_index, W1, a1s, a1d, b1, W2, a2s, a2d, b2,
           W3, a3s, a3d, b3, Wlin, blin):
    x_pad = jnp.pad(x, ((0, NPAD - N), (0, 0)))
    ei = edge_index.astype(jnp.int32)
    pad = jnp.full((EPAD - E,), PADN, jnp.int32)
    src = jnp.concatenate([ei[0], pad])
    dst = jnp.concatenate([ei[1], pad])
    epairs = jnp.stack([src.reshape(-1, CH), dst.reshape(-1, CH)], axis=1)

    b1r = b1.reshape(1, HID)
    b2r = b2.reshape(1, HID)
    b3r = b3.reshape(1, HID)
    blinr = blin.reshape(1, 4)

    h1pre, asad1 = _tc_layer1(x_pad, W1, a1s, a1d)
    acc1 = _sc_edges(h1pre, asad1, epairs)
    h1, h2pre, asad2 = _tc_mid(h1pre, asad1, acc1, b1r, W2, a2s, a2d)
    acc2 = _sc_edges(h2pre, asad2, epairs)
    h2, h3pre, asad3 = _tc_mid(h2pre, asad2, acc2, b2r, W3, a3s, a3d)
    acc3 = _sc_edges(h3pre, asad3, epairs)
    out = _tc_final(h1, h2, h3pre, asad3, acc3, b3r, Wlin, blinr)
    return out[:N]


# speedup vs baseline: 57.6106x; 1.0114x over previous
import dataclasses
import functools

import jax
import jax.numpy as jnp
from jax import lax
from jax.experimental import pallas as pl
from jax.experimental.pallas import tpu as pltpu
from jax.experimental.pallas import tpu_sc as plsc

N = 10000
E = 320000
F_IN = 128
HID = 16

NPAD = 10240
PADN = N
EPAD = 327680
NW = 32
EPW = EPAD // NW
CH = 128
NCHUNK = EPW // CH
ROWS_PT = NPAD // 16
ACCW = 32

_BLK = 1024


def _attn_cols(h, a_s_ref, a_d_ref):
    a_s = jnp.sum(h * a_s_ref[...], axis=1, keepdims=True)
    a_d = jnp.sum(h * a_d_ref[...], axis=1, keepdims=True)
    return jnp.concatenate([a_s, a_d], axis=1)


def _tc1_body(x_ref, w_ref, a_s_ref, a_d_ref, h_ref, asad_ref):
    h = jnp.dot(x_ref[...], w_ref[...], preferred_element_type=jnp.float32)
    h_ref[...] = h
    asad_ref[...] = _attn_cols(h, a_s_ref, a_d_ref)


def _tc_layer1(x_pad, W1, a1s, a1d):
    grid = (NPAD // _BLK,)
    return pl.pallas_call(
        _tc1_body,
        grid=grid,
        in_specs=[
            pl.BlockSpec((_BLK, F_IN), lambda i: (i, 0)),
            pl.BlockSpec((F_IN, HID), lambda i: (0, 0)),
            pl.BlockSpec((1, HID), lambda i: (0, 0)),
            pl.BlockSpec((1, HID), lambda i: (0, 0)),
        ],
        out_specs=[
            pl.BlockSpec((_BLK, HID), lambda i: (i, 0)),
            pl.BlockSpec((_BLK, 2), lambda i: (i, 0)),
        ],
        out_shape=[
            jax.ShapeDtypeStruct((NPAD, HID), jnp.float32),
            jax.ShapeDtypeStruct((NPAD, 2), jnp.float32),
        ],
    )(x_pad, W1, a1s, a1d)


def _post_aggregate(hpre_ref, asad_ref, acc_ref, b_ref):
    hpre = hpre_ref[...]
    asad = asad_ref[...]
    es = asad[:, 0:1] + asad[:, 1:2]
    ws = jnp.exp(jnp.where(es > 0, es, 0.2 * es))
    acc0 = acc_ref[0]
    acc1 = acc_ref[1]
    msg = acc0[:, 0:HID] + acc1[:, 0:HID] + ws * hpre
    den = acc0[:, HID:HID + 1] + acc1[:, HID:HID + 1] + ws
    g = msg / den + b_ref[...]
    g = jnp.where(g > 0, g, jnp.exp(jnp.minimum(g, 0.0)) - 1.0)
    nrm = jnp.sqrt(jnp.sum(g * g, axis=1, keepdims=True))
    return g / jnp.maximum(nrm, 1e-12)


def _tcmid_body(hpre_ref, asad_ref, acc_ref, b_ref, w_ref, a_s_ref, a_d_ref,
                hout_ref, hnext_ref, asadn_ref):
    hout = _post_aggregate(hpre_ref, asad_ref, acc_ref, b_ref)
    hout_ref[...] = hout
    hn = jnp.dot(hout, w_ref[...], preferred_element_type=jnp.float32)
    hnext_ref[...] = hn
    asadn_ref[...] = _attn_cols(hn, a_s_ref, a_d_ref)


def _tc_mid(hpre, asad, acc, b, Wn, ans, and_):
    grid = (NPAD // _BLK,)
    return pl.pallas_call(
        _tcmid_body,
        grid=grid,
        in_specs=[
            pl.BlockSpec((_BLK, HID), lambda i: (i, 0)),
            pl.BlockSpec((_BLK, 2), lambda i: (i, 0)),
            pl.BlockSpec((2, _BLK, ACCW), lambda i: (0, i, 0)),
            pl.BlockSpec((1, HID), lambda i: (0, 0)),
            pl.BlockSpec((HID, HID), lambda i: (0, 0)),
            pl.BlockSpec((1, HID), lambda i: (0, 0)),
            pl.BlockSpec((1, HID), lambda i: (0, 0)),
        ],
        out_specs=[
            pl.BlockSpec((_BLK, HID), lambda i: (i, 0)),
            pl.BlockSpec((_BLK, HID), lambda i: (i, 0)),
            pl.BlockSpec((_BLK, 2), lambda i: (i, 0)),
        ],
        out_shape=[
            jax.ShapeDtypeStruct((NPAD, HID), jnp.float32),
            jax.ShapeDtypeStruct((NPAD, HID), jnp.float32),
            jax.ShapeDtypeStruct((NPAD, 2), jnp.float32),
        ],
    )(hpre, asad, acc, b, Wn, ans, and_)


def _tcfin_body(h1_ref, h2_ref, hpre_ref, asad_ref, acc_ref, b_ref,
                wlin_ref, blin_ref, out_ref):
    h3 = _post_aggregate(hpre_ref, asad_ref, acc_ref, b_ref)
    emb = jnp.concatenate([h1_ref[...], h2_ref[...], h3], axis=1)
    wlin = wlin_ref[...]
    cols = [jnp.sum(emb * wlin[c:c + 1, :], axis=1, keepdims=True)
            for c in range(4)]
    logits = jnp.concatenate(cols, axis=1) + blin_ref[...]
    m = jnp.max(logits, axis=1, keepdims=True)
    lse = m + jnp.log(jnp.sum(jnp.exp(logits - m), axis=1, keepdims=True))
    out_ref[...] = logits - lse


def _tc_final(h1, h2, hpre, asad, acc, b, Wlin, blin):
    grid = (NPAD // _BLK,)
    return pl.pallas_call(
        _tcfin_body,
        grid=grid,
        in_specs=[
            pl.BlockSpec((_BLK, HID), lambda i: (i, 0)),
            pl.BlockSpec((_BLK, HID), lambda i: (i, 0)),
            pl.BlockSpec((_BLK, HID), lambda i: (i, 0)),
            pl.BlockSpec((_BLK, 2), lambda i: (i, 0)),
            pl.BlockSpec((2, _BLK, ACCW), lambda i: (0, i, 0)),
            pl.BlockSpec((1, HID), lambda i: (0, 0)),
            pl.BlockSpec((4, 3 * HID), lambda i: (0, 0)),
            pl.BlockSpec((1, 4), lambda i: (0, 0)),
        ],
        out_specs=pl.BlockSpec((_BLK, 4), lambda i: (i, 0)),
        out_shape=jax.ShapeDtypeStruct((NPAD, 4), jnp.float32),
    )(h1, h2, hpre, asad, acc, b, Wlin, blin)


def _sc_compiler_params():
    cp = pltpu.CompilerParams()
    if "needs_layout_passes" in pltpu.CompilerParams.__dataclass_fields__:
        cp = dataclasses.replace(cp, needs_layout_passes=False)
    if "use_tc_tiling_on_sc" in pltpu.CompilerParams.__dataclass_fields__:
        cp = dataclasses.replace(cp, use_tc_tiling_on_sc=False)
    return cp


GB = 2
NGRP = NCHUNK // GB
GE = GB * CH


def _sc_edges(h, asad, epairs):
    mesh = plsc.VectorSubcoreMesh(core_axis_name="c", subcore_axis_name="s")

    @functools.partial(
        pl.kernel,
        out_type=jax.ShapeDtypeStruct((2, NPAD, ACCW), jnp.float32),
        mesh=mesh,
        compiler_params=_sc_compiler_params(),
        scratch_types=[
            pltpu.VMEM((NPAD, 2), jnp.float32),
            [pltpu.VMEM((GB, 2, CH), jnp.int32) for _ in range(4)],
            pltpu.VMEM((GE, HID), jnp.float32),
            [pltpu.VMEM((GE, ACCW), jnp.float32) for _ in range(2)],
            pltpu.VMEM((GE,), jnp.float32),
            pltpu.VMEM_SHARED((NPAD, ACCW), jnp.float32),
            pltpu.SemaphoreType.DMA,
            pltpu.SemaphoreType.DMA,
            [pltpu.SemaphoreType.DMA for _ in range(2)],
        ],
    )
    def body(h_hbm, asad_hbm, ep_hbm, out_hbm,
             asad_v, idx_v, rows_v, srows_v, wbuf_v, acc_sh,
             isem, gsem, ssem):
        cid = lax.axis_index("c")
        sid = lax.axis_index("s")
        wid = cid * 16 + sid

        pltpu.sync_copy(asad_hbm, asad_v)

        z16 = jnp.zeros((16,), jnp.float32)

        for p in range(2):
            @pl.loop(0, GE)
            def _(r, p=p):
                srows_v[p][r, pl.ds(0, 16)] = z16
                srows_v[p][r, pl.ds(ACCW - 16, 16)] = z16

        @pl.loop(0, ROWS_PT // CH)
        def _(j):
            pltpu.sync_copy(srows_v[0].at[pl.ds(0, CH)],
                            acc_sh.at[pl.ds(sid * ROWS_PT + j * CH, CH)])

        plsc.subcore_barrier()

        iota16 = lax.iota(jnp.int32, 16)
        zero_i = jnp.zeros((16,), jnp.int32)
        one_i = jnp.full((16,), 1, jnp.int32)
        wcol_i = jnp.full((16,), 16, jnp.int32)
        cbase = wid * NCHUNK

        def idx_copy(gi, b):
            return pltpu.make_async_copy(
                ep_hbm.at[pl.ds(cbase + gi * GB, GB)], idx_v[b], isem)

        def gather_copy(b, p, j):
            return pltpu.make_async_copy(
                h_hbm.at[idx_v[b].at[j, 0]],
                rows_v.at[pl.ds(j * CH, CH)], gsem)

        def scatter_copy(b, p, j):
            return pltpu.async_copy(
                srows_v[p].at[pl.ds(j * CH, CH)],
                acc_sh.at[idx_v[b].at[j, 1]], ssem[p], add=True)

        def scatter_wait(b, p, j):
            pltpu.make_async_copy(
                srows_v[p].at[pl.ds(j * CH, CH)],
                acc_sh.at[idx_v[b].at[j, 1]], ssem[p]).wait()

        idx_copy(0, 0).start()

        @pl.loop(0, NGRP // 4)
        def _(s):
            for k in range(4):
                g = s * 4 + k
                b = k
                p = k % 2

                idx_copy(g, b).wait()
                for j in range(GB):
                    gather_copy(b, p, j).start()

                @plsc.parallel_loop(0, GE // 16, unroll=4)
                def _(t, b=b, p=p):
                    j = t // (CH // 16)
                    gg = t % (CH // 16)
                    src16 = idx_v[b][j, 0, pl.ds(gg * 16, 16)]
                    dst16 = idx_v[b][j, 1, pl.ds(gg * 16, 16)]
                    a_s = plsc.load_gather(asad_v, [src16, zero_i])
                    a_d = plsc.load_gather(asad_v, [dst16, one_i])
                    e = a_s + a_d
                    e = jnp.where(e > 0, e, 0.2 * e)
                    wbuf_v[pl.ds(t * 16, 16)] = jnp.exp(e)

                if k < 2:
                    @pl.when(s > 0)
                    def _(b=b, p=p):
                        for j in range(GB):
                            scatter_wait((b + 2) % 4, p, j)
                else:
                    for j in range(GB):
                        scatter_wait((b + 2) % 4, p, j)

                if k < 3:
                    idx_copy(g + 1, (b + 1) % 4).start()
                else:
                    @pl.when(s < NGRP // 4 - 1)
                    def _(g=g, b=b):
                        idx_copy(g + 1, (b + 1) % 4).start()

                for j in range(GB):
                    gather_copy(b, p, j).wait()

                @plsc.parallel_loop(0, GE // 16, unroll=4)
                def _(t, b=b, p=p):
                    o = t * 16
                    w = wbuf_v[pl.ds(o, 16)]
                    plsc.store_scatter(srows_v[p], [o + iota16, wcol_i], w)
                    for l in range(16):
                        wl = lax.gather(
                            w, jnp.full((16, 1), l, jnp.int32),
                            lax.GatherDimensionNumbers(
                                offset_dims=(), collapsed_slice_dims=(0,),
                                start_index_map=(0,)),
                            (1,),
                            mode=lax.GatherScatterMode.PROMISE_IN_BOUNDS)
                        srows_v[p][o + l, pl.ds(0, 16)] = \
                            rows_v[o + l, :] * wl

                for j in range(GB):
                    scatter_copy(b, p, j)

        for j in range(GB):
            scatter_wait(2, 0, j)
        for j in range(GB):
            scatter_wait(3, 1, j)

        plsc.subcore_barrier()

        pltpu.sync_copy(acc_sh.at[pl.ds(sid * ROWS_PT, ROWS_PT)],
                        out_hbm.at[cid, pl.ds(sid * ROWS_PT, ROWS_PT)])

    return body(h, asad, epairs)


def kernel(x, edge_index, W1, a1s, a1d, b1, W2, a2s, a2d, b2,
           W3, a3s, a3d, b3, Wlin, blin):
    x_pad = jnp.pad(x, ((0, NPAD - N), (0, 0)))
    ei = edge_index.astype(jnp.int32)
    pad = jnp.full((EPAD - E,), PADN, jnp.int32)
    src = jnp.concatenate([ei[0], pad])
    dst = jnp.concatenate([ei[1], pad])
    epairs = jnp.stack([src.reshape(-1, CH), dst.reshape(-1, CH)], axis=1)

    b1r = b1.reshape(1, HID)
    b2r = b2.reshape(1, HID)
    b3r = b3.reshape(1, HID)
    blinr = blin.reshape(1, 4)

    h1pre, asad1 = _tc_layer1(x_pad, W1, a1s, a1d)
    acc1 = _sc_edges(h1pre, asad1, epairs)
    h1, h2pre, asad2 = _tc_mid(h1pre, asad1, acc1, b1r, W2, a2s, a2d)
    acc2 = _sc_edges(h2pre, asad2, epairs)
    h2, h3pre, asad3 = _tc_mid(h2pre, asad2, acc2, b2r, W3, a3s, a3d)
    acc3 = _sc_edges(h3pre, asad3, epairs)
    out = _tc_final(h1, h2, h3pre, asad3, acc3, b3r, Wlin, blinr)
    return out[:N]
